# baseline (speedup 1.0000x reference)
import functools

import jax
import jax.numpy as jnp
from jax import lax
from jax.experimental import pallas as pl
from jax.experimental.pallas import tpu as pltpu
from jax.experimental.pallas import tpu_sc as plsc

B, F, V, D = 4096, 26, 2600000, 16
FD = F * D
MLP = [512, 256, 128]

NC, NS = 2, 16
NW = NC * NS
TOT = B * F
PER_W = TOT // NW
CH = 128
NCH = PER_W // CH

@functools.lru_cache(maxsize=1)
def _get_sc_gather():
    mesh = plsc.VectorSubcoreMesh(core_axis_name="c", subcore_axis_name="s")

    @functools.partial(
        pl.kernel,
        out_type=[
            jax.ShapeDtypeStruct((TOT, D), jnp.float32),
            jax.ShapeDtypeStruct((TOT,), jnp.float32),
        ],
        mesh=mesh,
        compiler_params=pltpu.CompilerParams(use_tc_tiling_on_sc=False),
        scratch_types=[
            pltpu.VMEM((NCH, CH), jnp.int32),
            pltpu.VMEM((PER_W, D), jnp.float32),
            pltpu.VMEM((PER_W,), jnp.float32),
            pltpu.SemaphoreType.DMA,
            pltpu.SemaphoreType.DMA,
        ],
    )
    def _sc_gather(x_hbm, emb_tbl, o1_tbl, emb_out, o1_out, idx_v, rows_v,
                   o1_v, sem_e, sem_o):
        wid = lax.axis_index("s") * NC + lax.axis_index("c")
        base = wid * PER_W

        pltpu.sync_copy(x_hbm.at[wid], idx_v)

        def chunk(i, carry):
            idx_row = idx_v.at[i]
            ce = pltpu.async_copy(emb_tbl.at[idx_row],
                                  rows_v.at[pl.ds(i * CH, CH)], sem_e)
            co = pltpu.async_copy(o1_tbl.at[idx_row],
                                  o1_v.at[pl.ds(i * CH, CH)], sem_o)
            ce.wait()
            co.wait()
            return carry

        lax.fori_loop(0, NCH, chunk, 0)

        pltpu.sync_copy(rows_v, emb_out.at[pl.ds(base, PER_W)])
        pltpu.sync_copy(o1_v, o1_out.at[pl.ds(base, PER_W)])

    return _sc_gather


def _tc_body(emb_ref, o1_ref, w1_ref, b1_ref, g1_ref, be1_ref,
             w2_ref, b2_ref, g2_ref, be2_ref, w3_ref, b3_ref,
             w4_ref, b4_ref, out_ref):
    emb = emb_ref[...]
    o1 = jnp.sum(o1_ref[...], axis=1, keepdims=True)

    ki = lax.broadcasted_iota(jnp.int32, (FD, D), 0)
    di = lax.broadcasted_iota(jnp.int32, (FD, D), 1)
    S = (ki % D == di).astype(jnp.float32)
    sums = jnp.dot(emb, S, preferred_element_type=jnp.float32)
    sq_of_sum = jnp.sum(sums * sums, axis=1, keepdims=True)
    sum_of_sq = jnp.sum(emb * emb, axis=1, keepdims=True)
    o2 = 0.5 * (sq_of_sum - sum_of_sq)

    def bn_relu(h, g, be):
        m = jnp.mean(h, axis=0, keepdims=True)
        v = jnp.mean((h - m) * (h - m), axis=0, keepdims=True)
        return jnp.maximum((h - m) / jnp.sqrt(v + 1e-5) * g + be, 0.0)

    h = jnp.dot(emb, w1_ref[...], preferred_element_type=jnp.float32)
    h = bn_relu(h + b1_ref[...], g1_ref[...], be1_ref[...])
    h = jnp.dot(h, w2_ref[...], preferred_element_type=jnp.float32)
    h = bn_relu(h + b2_ref[...], g2_ref[...], be2_ref[...])
    h = jnp.dot(h, w3_ref[...], preferred_element_type=jnp.float32) + b3_ref[...]
    dnn = jnp.dot(h, w4_ref[...], preferred_element_type=jnp.float32) + b4_ref[...]

    out_ref[...] = o1 + o2 + dnn


_tc_call = pl.pallas_call(
    _tc_body,
    out_shape=jax.ShapeDtypeStruct((B, 1), jnp.float32),
)


def kernel(x, cat_embed, o1_fc, W1, b1, g1, be1, W2, b2, g2, be2, W3, b3,
           W4, b4):
    x3d = x.astype(jnp.int32).reshape(NW, NCH, CH)
    emb_flat, o1_flat = _get_sc_gather()(x3d, cat_embed, o1_fc.reshape(V))
    emb = emb_flat.reshape(B, FD)
    o1v = o1_flat.reshape(B, F)
    return _tc_call(emb, o1v, W1.T, b1.reshape(1, -1), g1.reshape(1, -1),
                    be1.reshape(1, -1), W2.T, b2.reshape(1, -1),
                    g2.reshape(1, -1), be2.reshape(1, -1), W3.T,
                    b3.reshape(1, -1), W4.T, b4.reshape(1, -1))
